# baseline (speedup 1.0000x reference)
import jax
import jax.numpy as jnp
from jax import lax
from jax.experimental import pallas as pl
from jax.experimental.pallas import tpu as pltpu
from jax.experimental.pallas import tpu_sc as plsc

N = 10000
E = 320000
IN = 128
OUT = 128
R = 8

N_PAD = 10240
C = 128
NW = 32
NCHUNK = 80
EPT = NCHUNK * C
E_PAD = NW * EPT
MROWS = NW * NCHUNK
N_ACC = 10240
ROWS_PER_TILE = N_ACC // 16
MM_BLK = 512


def _mm_body(x_ref, w_ref, src_ref, rel_ref, o_ref, gidx_ref):
    o_ref[0] = jnp.dot(x_ref[...], w_ref[0], preferred_element_type=jnp.float32)
    gidx_ref[...] = rel_ref[...] * N_PAD + src_ref[...]


def _transform(x_pad, weight, src_p, rel_p):
    nblk = N_PAD // MM_BLK
    grows = MROWS // (R * nblk)
    return pl.pallas_call(
        _mm_body,
        grid=(R, nblk),
        in_specs=[
            pl.BlockSpec((MM_BLK, IN), lambda r, i: (i, 0)),
            pl.BlockSpec((1, IN, OUT), lambda r, i: (r, 0, 0)),
            pl.BlockSpec((grows, C), lambda r, i: (r * nblk + i, 0)),
            pl.BlockSpec((grows, C), lambda r, i: (r * nblk + i, 0)),
        ],
        out_specs=[
            pl.BlockSpec((1, MM_BLK, OUT), lambda r, i: (r, i, 0)),
            pl.BlockSpec((grows, C), lambda r, i: (r * nblk + i, 0)),
        ],
        out_shape=[
            jax.ShapeDtypeStruct((R, N_PAD, OUT), jnp.float32),
            jax.ShapeDtypeStruct((MROWS, C), jnp.int32),
        ],
    )(x_pad, weight, src_p, rel_p)


def _add_body(a_ref, b_ref, o_ref):
    o_ref[...] = a_ref[0] + b_ref[0]


def _combine(partials):
    blk = 2000
    return pl.pallas_call(
        _add_body,
        grid=(N // blk,),
        in_specs=[
            pl.BlockSpec((1, blk, OUT), lambda i: (0, i, 0)),
            pl.BlockSpec((1, blk, OUT), lambda i: (1, i, 0)),
        ],
        out_specs=pl.BlockSpec((blk, OUT), lambda i: (i, 0)),
        out_shape=jax.ShapeDtypeStruct((N, OUT), jnp.float32),
    )(partials, partials)


def _sc_body(t_hbm, gidx_hbm, dst_hbm, norm_hbm, zeros_hbm, out_hbm,
             gidx2d, dst2, norm2, rows_a, rows_b, accum,
             sem_ga, sem_gb, sem_ma, sem_mb):
    cid = lax.axis_index("c")
    sid = lax.axis_index("s")
    w = cid * 16 + sid

    pltpu.sync_copy(zeros_hbm, accum.at[pl.ds(sid * ROWS_PER_TILE, ROWS_PER_TILE)])
    pltpu.sync_copy(gidx_hbm.at[pl.ds(w * NCHUNK, NCHUNK)], gidx2d)

    def _meta(t, slot, sem):
        pltpu.async_copy(dst_hbm.at[w * NCHUNK + t], dst2.at[slot], sem)
        pltpu.async_copy(norm_hbm.at[w * NCHUNK + t], norm2.at[slot], sem)

    def _meta_wait(slot, sem):
        pltpu.make_async_copy(dst_hbm.at[0], dst2.at[slot], sem).wait()
        pltpu.make_async_copy(norm_hbm.at[0], norm2.at[slot], sem).wait()

    _meta(0, 0, sem_ma)
    _meta(1, 1, sem_mb)
    with jax.named_scope("sc_zero_barrier"):
        plsc.subcore_barrier()

    def _gather(t, rows, sem):
        pltpu.async_copy(t_hbm.at[gidx2d.at[t]], rows, sem)

    _gather(0, rows_a, sem_ga)

    def _half(t, rows, rows_nxt, sem, sem_nxt, slot, msem):
        pltpu.make_async_copy(t_hbm.at[gidx2d.at[t]], rows, sem).wait()
        @pl.when(t + 1 < NCHUNK)
        def _():
            _gather(t + 1, rows_nxt, sem_nxt)
        _meta_wait(slot, msem)
        @pl.loop(0, C // 16)
        def _grp(g):
            nv = norm2[slot, pl.ds(g * 16, 16)]
            for i in range(16):
                nb = nv[i]
                for c8 in range(OUT // 16):
                    csl = pl.ds(c8 * 16, 16)
                    rows[g * 16 + i, csl] = rows[g * 16 + i, csl] * nb
        pltpu.sync_copy(rows, accum.at[dst2.at[slot]], add=True)
        @pl.when(t + 2 < NCHUNK)
        def _():
            _meta(t + 2, slot, msem)

    with jax.named_scope("sc_edge_loop"):
        @pl.loop(0, NCHUNK // 2)
        def _pair(k):
            _half(2 * k, rows_a, rows_b, sem_ga, sem_gb, 0, sem_ma)
            _half(2 * k + 1, rows_b, rows_a, sem_gb, sem_ga, 1, sem_mb)

    with jax.named_scope("sc_dump"):
        plsc.subcore_barrier()
        orows = pl.ds(sid * ROWS_PER_TILE, ROWS_PER_TILE)
        pltpu.sync_copy(accum.at[orows], out_hbm.at[cid, orows])


def _sc_edge_pass(t_flat, gidx_p, dst_p, norm_p, zeros):
    mesh = plsc.VectorSubcoreMesh(core_axis_name="c", subcore_axis_name="s")
    return pl.kernel(
        _sc_body,
        out_type=jax.ShapeDtypeStruct((2, N_ACC, OUT), jnp.float32),
        mesh=mesh,
        scratch_types=[
            pltpu.VMEM((NCHUNK, C), jnp.int32),
            pltpu.VMEM((2, C), jnp.int32),
            pltpu.VMEM((2, C), jnp.float32),
            pltpu.VMEM((C, OUT), jnp.float32),
            pltpu.VMEM((C, OUT), jnp.float32),
            pltpu.VMEM_SHARED((N_ACC, OUT), jnp.float32),
            pltpu.SemaphoreType.DMA,
            pltpu.SemaphoreType.DMA,
            pltpu.SemaphoreType.DMA,
            pltpu.SemaphoreType.DMA,
        ],
    )(t_flat, gidx_p, dst_p, norm_p, zeros)


def kernel(x, weight, norm, edge_index, rel_type):
    src = edge_index[0]
    dst = edge_index[1]
    norm_f = norm[:, 0]
    pad = E_PAD - E
    src_p = jnp.pad(src, (0, pad)).reshape(MROWS, C)
    dst_p = jnp.pad(dst, (0, pad)).reshape(MROWS, C)
    rel_p = jnp.pad(rel_type, (0, pad)).reshape(MROWS, C)
    norm_p = jnp.pad(norm_f, (0, pad)).reshape(MROWS, C)
    x_pad = jnp.pad(x, ((0, N_PAD - N), (0, 0)))

    t, gidx_p = _transform(x_pad, weight, src_p, rel_p)
    t_flat = t.reshape(R * N_PAD, OUT)
    zeros = jnp.zeros((ROWS_PER_TILE, OUT), jnp.float32)
    partials = _sc_edge_pass(t_flat, gidx_p, dst_p, norm_p, zeros)
    return _combine(partials)


# speedup vs baseline: 1.8866x; 1.8866x over previous
import jax
import jax.numpy as jnp
from jax import lax
from jax.experimental import pallas as pl
from jax.experimental.pallas import tpu as pltpu
from jax.experimental.pallas import tpu_sc as plsc

N = 10000
E = 320000
IN = 128
OUT = 128
R = 8

N_PAD = 10240
C = 128
NW = 32
NCHUNK = 80
EPT = NCHUNK * C
E_PAD = NW * EPT
MROWS = NW * NCHUNK
N_ACC = 10240
ROWS_PER_TILE = N_ACC // 16
MM_BLK = 512


def _mm_body(x_ref, w_ref, src_ref, rel_ref, o_ref, gidx_ref):
    o_ref[0] = jnp.dot(x_ref[...], w_ref[0], preferred_element_type=jnp.float32)
    gidx_ref[...] = rel_ref[...] * N_PAD + src_ref[...]


def _transform(x_pad, weight, src_p, rel_p):
    nblk = N_PAD // MM_BLK
    grows = MROWS // (R * nblk)
    return pl.pallas_call(
        _mm_body,
        grid=(R, nblk),
        in_specs=[
            pl.BlockSpec((MM_BLK, IN), lambda r, i: (i, 0)),
            pl.BlockSpec((1, IN, OUT), lambda r, i: (r, 0, 0)),
            pl.BlockSpec((grows, C), lambda r, i: (r * nblk + i, 0)),
            pl.BlockSpec((grows, C), lambda r, i: (r * nblk + i, 0)),
        ],
        out_specs=[
            pl.BlockSpec((1, MM_BLK, OUT), lambda r, i: (r, i, 0)),
            pl.BlockSpec((grows, C), lambda r, i: (r * nblk + i, 0)),
        ],
        out_shape=[
            jax.ShapeDtypeStruct((R, N_PAD, OUT), jnp.float32),
            jax.ShapeDtypeStruct((MROWS, C), jnp.int32),
        ],
    )(x_pad, weight, src_p, rel_p)


def _add_body(a_ref, b_ref, o_ref):
    o_ref[...] = a_ref[0] + b_ref[0]


def _combine(partials):
    blk = 2000
    return pl.pallas_call(
        _add_body,
        grid=(N // blk,),
        in_specs=[
            pl.BlockSpec((1, blk, OUT), lambda i: (0, i, 0)),
            pl.BlockSpec((1, blk, OUT), lambda i: (1, i, 0)),
        ],
        out_specs=pl.BlockSpec((blk, OUT), lambda i: (i, 0)),
        out_shape=jax.ShapeDtypeStruct((N, OUT), jnp.float32),
    )(partials, partials)


def _sc_body(t_hbm, gidx_hbm, dst_hbm, norm_hbm, zeros_hbm, out_hbm,
             gidx2d, dst2, norm2, rows_a, rows_b, accum,
             sem_ga, sem_gb, sem_ma, sem_mb):
    cid = lax.axis_index("c")
    sid = lax.axis_index("s")
    w = cid * 16 + sid

    pltpu.sync_copy(zeros_hbm, accum.at[pl.ds(sid * ROWS_PER_TILE, ROWS_PER_TILE)])
    pltpu.sync_copy(gidx_hbm.at[pl.ds(w * NCHUNK, NCHUNK)], gidx2d)

    def _meta(t, slot, sem):
        pltpu.async_copy(dst_hbm.at[w * NCHUNK + t], dst2.at[slot], sem)
        pltpu.async_copy(norm_hbm.at[w * NCHUNK + t], norm2.at[slot], sem)

    def _meta_wait(slot, sem):
        pltpu.make_async_copy(dst_hbm.at[0], dst2.at[slot], sem).wait()
        pltpu.make_async_copy(norm_hbm.at[0], norm2.at[slot], sem).wait()

    _meta(0, 0, sem_ma)
    _meta(1, 1, sem_mb)
    with jax.named_scope("sc_zero_barrier"):
        plsc.subcore_barrier()

    def _gather(t, rows, sem):
        pltpu.async_copy(t_hbm.at[gidx2d.at[t]], rows, sem)

    _gather(0, rows_a, sem_ga)

    def _half(t, rows, rows_nxt, sem, sem_nxt, slot, msem):
        pltpu.make_async_copy(t_hbm.at[gidx2d.at[t]], rows, sem).wait()
        @pl.when(t + 1 < NCHUNK)
        def _():
            _gather(t + 1, rows_nxt, sem_nxt)
        _meta_wait(slot, msem)
        @pl.loop(0, C // 16)
        def _grp(g):
            nv = norm2[slot, pl.ds(g * 16, 16)]
            for i in range(16):
                nb = nv[i]
                for c8 in range(OUT // 16):
                    csl = pl.ds(c8 * 16, 16)
                    rows[g * 16 + i, csl] = rows[g * 16 + i, csl] * nb
        pltpu.sync_copy(rows, accum.at[dst2.at[slot]], add=True)
        @pl.when(t + 2 < NCHUNK)
        def _():
            _meta(t + 2, slot, msem)

    with jax.named_scope("sc_edge_loop"):
        @pl.loop(0, NCHUNK // 2)
        def _pair(k):
            _half(2 * k, rows_a, rows_b, sem_ga, sem_gb, 0, sem_ma)
            _half(2 * k + 1, rows_b, rows_a, sem_gb, sem_ga, 1, sem_mb)

    with jax.named_scope("sc_dump"):
        plsc.subcore_barrier()
        orows = pl.ds(sid * ROWS_PER_TILE, ROWS_PER_TILE)
        pltpu.sync_copy(accum.at[orows], out_hbm.at[cid, orows])


def _sc_edge_pass(t_flat, gidx_p, dst_p, norm_p, zeros):
    mesh = plsc.VectorSubcoreMesh(core_axis_name="c", subcore_axis_name="s")
    return pl.kernel(
        _sc_body,
        out_type=jax.ShapeDtypeStruct((2, N_ACC, OUT), jnp.float32),
        mesh=mesh,
        scratch_types=[
            pltpu.VMEM((NCHUNK, C), jnp.int32),
            pltpu.VMEM((2, C), jnp.int32),
            pltpu.VMEM((2, C), jnp.float32),
            pltpu.VMEM((C, OUT), jnp.float32),
            pltpu.VMEM((C, OUT), jnp.float32),
            pltpu.VMEM_SHARED((N_ACC, OUT), jnp.float32),
            pltpu.SemaphoreType.DMA,
            pltpu.SemaphoreType.DMA,
            pltpu.SemaphoreType.DMA,
            pltpu.SemaphoreType.DMA,
        ],
    )(t_flat, gidx_p, dst_p, norm_p, zeros)


def kernel(x, weight, norm, edge_index, rel_type):
    src = edge_index[0]
    dst = edge_index[1]
    norm_f = norm[:, 0]
    pad = E_PAD - E
    pad_idx = jnp.arange(pad, dtype=jnp.int32)
    src_p = jnp.concatenate([src, pad_idx % N]).reshape(MROWS, C)
    dst_p = jnp.concatenate([dst, N + pad_idx % (N_ACC - N)]).reshape(MROWS, C)
    rel_p = jnp.pad(rel_type, (0, pad)).reshape(MROWS, C)
    norm_p = jnp.pad(norm_f, (0, pad)).reshape(MROWS, C)
    x_pad = jnp.pad(x, ((0, N_PAD - N), (0, 0)))

    t, gidx_p = _transform(x_pad, weight, src_p, rel_p)
    t_flat = t.reshape(R * N_PAD, OUT)
    zeros = jnp.zeros((ROWS_PER_TILE, OUT), jnp.float32)
    partials = _sc_edge_pass(t_flat, gidx_p, dst_p, norm_p, zeros)
    return _combine(partials)
